# baseline (speedup 1.0000x reference)
import jax
import jax.numpy as jnp
from jax import lax
from jax.experimental import pallas as pl

E = 8
K = 2
H = 768
BT = 4096


def _router_body(x_ref, w_ref, bc_ref, lot_ref, rwt_ref, set_ref):
    x = x_ref[...]
    w = w_ref[...]
    dn = (((1,), (1,)), ((), ()))
    lot = (
        lax.dot_general(w, x, dn, preferred_element_type=jnp.float32)
        + bc_ref[...]
    )
    lot_ref[...] = lot
    iota = lax.broadcasted_iota(jnp.int32, lot.shape, 0)
    m1 = jnp.max(lot, axis=0, keepdims=True)
    a1 = jnp.min(jnp.where(lot == m1, iota, E), axis=0, keepdims=True)
    masked = jnp.where(iota == a1, -jnp.inf, lot)
    m2 = jnp.max(masked, axis=0, keepdims=True)
    a2 = jnp.min(jnp.where(masked == m2, iota, E), axis=0, keepdims=True)
    p = jnp.exp(m2 - m1)
    denom = p + 1.0
    rwt_ref[...] = jnp.concatenate([1.0 / denom, p / denom], axis=0)
    set_ref[...] = jnp.concatenate([a1, a2], axis=0)


def _make_router_call(T):
    return pl.pallas_call(
        _router_body,
        grid=(T // BT,),
        in_specs=[
            pl.BlockSpec((BT, H), lambda i: (i, 0)),
            pl.BlockSpec((E, H), lambda i: (0, 0)),
            pl.BlockSpec((E, 1), lambda i: (0, 0)),
        ],
        out_specs=[
            pl.BlockSpec((E, BT), lambda i: (0, i)),
            pl.BlockSpec((K, BT), lambda i: (0, i)),
            pl.BlockSpec((K, BT), lambda i: (0, i)),
        ],
        out_shape=[
            jax.ShapeDtypeStruct((E, T), jnp.float32),
            jax.ShapeDtypeStruct((K, T), jnp.float32),
            jax.ShapeDtypeStruct((K, T), jnp.int32),
        ],
    )


def kernel(hidden_states, W, b):
    bsz, seq, hid = hidden_states.shape
    T = bsz * seq
    x = hidden_states.reshape(T, hid)
    bc = b.reshape(E, 1)
    lot, rwt, sett = _make_router_call(T)(x, W, bc)
    return rwt.T.reshape(bsz, seq, K), sett.T.reshape(bsz, seq, K), lot.T
